# baseline (speedup 1.0000x reference)
import functools

import jax
import jax.numpy as jnp
from jax import lax
from jax.experimental import pallas as pl
from jax.experimental.pallas import tpu as pltpu
from jax.experimental.pallas import tpu_sc as plsc

_VOCAB = 1_000_000
_DIM = 64
_BATCH = 16384
_NNEG = 20
_NC = 2
_NS = 16
_NW = _NC * _NS
_BPW = _BATCH // _NW
_CB = 32
_NCHUNK = _BPW // _CB
_NROW = 1 + _NNEG
_W = 2 * _DIM


def _sc_scores(center_idx, pos_idx, neg_idx, in_emb2, out_emb2):
    mesh = plsc.VectorSubcoreMesh(core_axis_name="c", subcore_axis_name="s")

    @functools.partial(
        pl.kernel,
        mesh=mesh,
        out_type=jax.ShapeDtypeStruct((_NW, _NROW * _BPW), jnp.float32),
        compiler_params=pltpu.CompilerParams(
            needs_layout_passes=False, use_tc_tiling_on_sc=True),
        scratch_types=[
            pltpu.VMEM((_BPW,), jnp.int32),
            pltpu.VMEM((_BPW,), jnp.int32),
            pltpu.VMEM((_BPW * _NNEG,), jnp.int32),
            pltpu.VMEM((_BPW,), jnp.int32),
            pltpu.VMEM((_BPW,), jnp.int32),
            pltpu.VMEM((_BPW * _NNEG,), jnp.int32),
            pltpu.VMEM((_CB, _W), jnp.float32),
            pltpu.VMEM((_CB, _W), jnp.float32),
            pltpu.VMEM((_CB * _NNEG // 128, 128, _W), jnp.float32),
            pltpu.VMEM((_NROW * _BPW,), jnp.float32),
            pltpu.VMEM((_DIM * 16,), jnp.float32),
            pltpu.SemaphoreType.DMA,
        ],
    )
    def scores_kernel(center_hbm, pos_hbm, neg_hbm, in_hbm, out_hbm,
                      scores_hbm, idx_c, idx_p, idx_n, cb_c, cb_p, cb_n,
                      crow, prow, nrow, sbuf, cT, sem):
        wid = lax.axis_index("s") * _NC + lax.axis_index("c")
        base = wid * _BPW
        pltpu.sync_copy(center_hbm.at[pl.ds(base, _BPW)], idx_c)
        pltpu.sync_copy(pos_hbm.at[pl.ds(base, _BPW)], idx_p)
        pltpu.sync_copy(neg_hbm.at[pl.ds(base * _NNEG, _BPW * _NNEG)], idx_n)

        def split_pass(idx_ref, cb_ref, n16):
            def body(i, carry):
                sl = pl.ds(i * 16, 16)
                v = idx_ref[sl]
                idx_ref[sl] = lax.shift_right_logical(v, 1)
                cb_ref[sl] = lax.shift_left(lax.bitwise_and(v, 1), 6)
                return carry
            lax.fori_loop(0, n16, body, 0, unroll=8)

        split_pass(idx_c, cb_c, _BPW // 16)
        split_pass(idx_p, cb_p, _BPW // 16)
        split_pass(idx_n, cb_n, _BPW * _NNEG // 16)

        lane = lax.iota(jnp.int32, 16)

        def chunk_body(t, carry):
            off = t * _CB
            cps = [
                pltpu.async_copy(in_hbm.at[idx_c.at[pl.ds(off, _CB)]], crow, sem),
                pltpu.async_copy(out_hbm.at[idx_p.at[pl.ds(off, _CB)]], prow, sem),
            ]
            for j in range(_CB * _NNEG // 128):
                cps.append(pltpu.async_copy(
                    out_hbm.at[idx_n.at[pl.ds(off * _NNEG + j * 128, 128)]],
                    nrow.at[j], sem))
            for cp in cps:
                cp.wait()
            zero = jnp.zeros((16,), jnp.float32)
            for g in range(_CB // 16):
                rows = lane + g * 16
                gpos = rows + off
                nbase = rows * _NNEG
                colc = plsc.load_gather(cb_c, [gpos])

                def trans_body(d, c_):
                    dvec = jnp.zeros((16,), jnp.int32) + d
                    cT[pl.ds(d * 16, 16)] = plsc.load_gather(
                        crow, [rows, dvec + colc])
                    return c_

                lax.fori_loop(0, _DIM, trans_body, 0, unroll=8)
                for k0 in range(0, _NROW, 7):
                    ks = range(k0, min(k0 + 7, _NROW))
                    cols, js, rs = [], [], []
                    for k in ks:
                        if k == 0:
                            cols.append(plsc.load_gather(cb_p, [gpos]))
                            js.append(None)
                            rs.append(rows)
                        else:
                            cols.append(plsc.load_gather(
                                cb_n, [gpos * _NNEG + (k - 1)]))
                            rid = nbase + (k - 1)
                            js.append(lax.shift_right_logical(rid, 7))
                            rs.append(lax.bitwise_and(rid, 127))

                    def dot_body(d, accs):
                        dvec = jnp.zeros((16,), jnp.int32) + d
                        c = cT[pl.ds(d * 16, 16)]
                        out = []
                        for i, k in enumerate(ks):
                            if k == 0:
                                x = plsc.load_gather(
                                    prow, [rs[i], dvec + cols[i]])
                            else:
                                x = plsc.load_gather(
                                    nrow, [js[i], rs[i], dvec + cols[i]])
                            out.append(accs[i] + c * x)
                        return tuple(out)

                    res = lax.fori_loop(0, _DIM, dot_body,
                                        (zero,) * len(ks), unroll=8)
                    for i, k in enumerate(ks):
                        sl = pl.ds(k * _BPW + off + g * 16, 16)
                        sbuf[sl] = res[i] if k == 0 else -res[i]
            return carry

        lax.fori_loop(0, _NCHUNK, chunk_body, 0)
        pltpu.sync_copy(sbuf, scores_hbm.at[wid])

    return scores_kernel(center_idx, pos_idx, neg_idx, in_emb2, out_emb2)


def _loss_tc(scores_flat):
    def body(x_ref, o_ref):
        o_ref[0, 0] = -jnp.sum(jax.nn.log_sigmoid(x_ref[...])) / _BATCH

    return pl.pallas_call(
        body,
        out_shape=jax.ShapeDtypeStruct((1, 1), jnp.float32),
        out_specs=pl.BlockSpec(memory_space=pltpu.SMEM),
    )(scores_flat)


def kernel(center_words, pos_context_words, neg_context_words, in_emb, out_emb):
    c = center_words.astype(jnp.int32)
    p = pos_context_words.astype(jnp.int32)
    n = neg_context_words.astype(jnp.int32).reshape(-1)
    in2 = in_emb.reshape(_VOCAB // 2, _W)
    out2 = out_emb.reshape(_VOCAB // 2, _W)
    scores = _sc_scores(c, p, n, in2, out2)
    loss = _loss_tc(scores.reshape(_NW * _NROW, _BPW))
    return loss[0, 0]
